# baseline (speedup 1.0000x reference)
import functools

import jax
import jax.numpy as jnp
from jax import lax
from jax.experimental import pallas as pl
from jax.experimental.pallas import tpu as pltpu
from jax.experimental.pallas import tpu_sc as plsc

_NUM_CORES = 2
_NUM_SUBCORES = 16
_NW = _NUM_CORES * _NUM_SUBCORES

_CHUNK = 128
_NBUF = 4


@functools.cache
def _build(V, D, S):
    mesh = plsc.VectorSubcoreMesh(core_axis_name="c", subcore_axis_name="s")

    @functools.partial(
        pl.kernel,
        out_type=jax.ShapeDtypeStruct((_NW, S, _CHUNK, D), jnp.float32),
        mesh=mesh,
        compiler_params=pltpu.CompilerParams(use_tc_tiling_on_sc=False),
        scratch_types=[
            pltpu.VMEM((S, _CHUNK), jnp.int32),
            pltpu.VMEM((_NBUF, _CHUNK, D), jnp.float32),
            [pltpu.SemaphoreType.DMA] * _NBUF,
            [pltpu.SemaphoreType.DMA] * _NBUF,
        ],
    )
    def gather_kernel(idx_hbm, table_hbm, out_hbm, idx_v, rows_v, gsems, wsems):
        wid = lax.axis_index("s") * _NUM_CORES + lax.axis_index("c")

        pltpu.sync_copy(idx_hbm.at[wid], idx_v)

        def start_gather(c, slot):
            pltpu.async_copy(table_hbm.at[idx_v.at[c]], rows_v.at[slot],
                             gsems[slot])

        def wait_gather(c, slot):
            pltpu.make_async_copy(table_hbm.at[idx_v.at[c]], rows_v.at[slot],
                                  gsems[slot]).wait()

        def start_write(c, slot):
            pltpu.async_copy(rows_v.at[slot], out_hbm.at[wid, c], wsems[slot])

        def wait_write(c, slot):
            pltpu.make_async_copy(rows_v.at[slot], out_hbm.at[wid, c],
                                  wsems[slot]).wait()

        for b in range(_NBUF):
            start_gather(b, b)

        def body(g, _):
            for b in range(_NBUF):
                c = g * _NBUF + b
                wait_gather(c, b)
                start_write(c, b)
                wait_write(c, b)
                start_gather(c + _NBUF, b)
            return _

        lax.fori_loop(0, S // _NBUF - 1, body, None)

        for b in range(_NBUF):
            c = S - _NBUF + b
            wait_gather(c, b)
            start_write(c, b)
        for b in range(_NBUF):
            wait_write(S - _NBUF + b, b)

    return gather_kernel


def kernel(X, table):
    V, D = table.shape
    B = X.size
    assert B % (_NW * _CHUNK) == 0
    S = B // (_NW * _CHUNK)
    idx = X.reshape(_NW, S, _CHUNK).astype(jnp.int32)
    out = _build(V, D, S)(idx, table)
    return out.reshape(X.shape + (D,))


# speedup vs baseline: 1.0007x; 1.0007x over previous
import functools

import jax
import jax.numpy as jnp
from jax import lax
from jax.experimental import pallas as pl
from jax.experimental.pallas import tpu as pltpu
from jax.experimental.pallas import tpu_sc as plsc

_NUM_CORES = 2
_NUM_SUBCORES = 16
_NW = _NUM_CORES * _NUM_SUBCORES

_CHUNK = 512
_NBUF = 2


@functools.cache
def _build(V, D, S):
    mesh = plsc.VectorSubcoreMesh(core_axis_name="c", subcore_axis_name="s")

    @functools.partial(
        pl.kernel,
        out_type=jax.ShapeDtypeStruct((_NW, S, _CHUNK, D), jnp.float32),
        mesh=mesh,
        compiler_params=pltpu.CompilerParams(use_tc_tiling_on_sc=False),
        scratch_types=[
            pltpu.VMEM((S, _CHUNK), jnp.int32),
            pltpu.VMEM((_NBUF, _CHUNK, D), jnp.float32),
            [pltpu.SemaphoreType.DMA] * _NBUF,
            [pltpu.SemaphoreType.DMA] * _NBUF,
        ],
    )
    def gather_kernel(idx_hbm, table_hbm, out_hbm, idx_v, rows_v, gsems, wsems):
        wid = lax.axis_index("s") * _NUM_CORES + lax.axis_index("c")

        pltpu.sync_copy(idx_hbm.at[wid], idx_v)

        def start_gather(c, slot):
            pltpu.async_copy(table_hbm.at[idx_v.at[c]], rows_v.at[slot],
                             gsems[slot])

        def wait_gather(c, slot):
            pltpu.make_async_copy(table_hbm.at[idx_v.at[c]], rows_v.at[slot],
                                  gsems[slot]).wait()

        def start_write(c, slot):
            pltpu.async_copy(rows_v.at[slot], out_hbm.at[wid, c], wsems[slot])

        def wait_write(c, slot):
            pltpu.make_async_copy(rows_v.at[slot], out_hbm.at[wid, c],
                                  wsems[slot]).wait()

        for b in range(_NBUF):
            start_gather(b, b)

        def body(g, _):
            for b in range(_NBUF):
                c = g * _NBUF + b
                wait_gather(c, b)
                start_write(c, b)
                wait_write(c, b)
                start_gather(c + _NBUF, b)
            return _

        lax.fori_loop(0, S // _NBUF - 1, body, None)

        for b in range(_NBUF):
            c = S - _NBUF + b
            wait_gather(c, b)
            start_write(c, b)
        for b in range(_NBUF):
            wait_write(S - _NBUF + b, b)

    return gather_kernel


def kernel(X, table):
    V, D = table.shape
    B = X.size
    assert B % (_NW * _CHUNK) == 0
    S = B // (_NW * _CHUNK)
    idx = X.reshape(_NW, S, _CHUNK).astype(jnp.int32)
    out = _build(V, D, S)(idx, table)
    return out.reshape(X.shape + (D,))


# speedup vs baseline: 1.3291x; 1.3282x over previous
import functools

import jax
import jax.numpy as jnp
from jax import lax
from jax.experimental import pallas as pl
from jax.experimental.pallas import tpu as pltpu
from jax.experimental.pallas import tpu_sc as plsc

_NUM_CORES = 2
_NUM_SUBCORES = 16
_NW = _NUM_CORES * _NUM_SUBCORES

_CHUNK = 128
_NBUF = 4


@functools.cache
def _build(V, D, S):
    mesh = plsc.VectorSubcoreMesh(core_axis_name="c", subcore_axis_name="s")

    @functools.partial(
        pl.kernel,
        out_type=jax.ShapeDtypeStruct((_NW, S, _CHUNK, 2 * D), jnp.float32),
        mesh=mesh,
        compiler_params=pltpu.CompilerParams(use_tc_tiling_on_sc=False),
        scratch_types=[
            pltpu.VMEM((S, _CHUNK), jnp.int32),
            pltpu.VMEM((_NBUF, _CHUNK, D), jnp.float32),
            [pltpu.SemaphoreType.DMA] * _NBUF,
            [pltpu.SemaphoreType.DMA] * _NBUF,
        ],
    )
    def gather_kernel(idx_hbm, table_hbm, out_hbm, idx_v, rows_v, gsems, wsems):
        wid = lax.axis_index("s") * _NUM_CORES + lax.axis_index("c")

        pltpu.sync_copy(idx_hbm.at[wid], idx_v)

        def start_gather(c, slot):
            pltpu.async_copy(table_hbm.at[idx_v.at[c]], rows_v.at[slot],
                             gsems[slot])

        def wait_gather(c, slot):
            pltpu.make_async_copy(table_hbm.at[idx_v.at[c]], rows_v.at[slot],
                                  gsems[slot]).wait()

        def start_write(c, slot):
            pltpu.async_copy(rows_v.at[slot],
                             out_hbm.at[wid, c, :, pl.ds(0, D)], wsems[slot])

        def wait_write(c, slot):
            pltpu.make_async_copy(rows_v.at[slot],
                                  out_hbm.at[wid, c, :, pl.ds(0, D)],
                                  wsems[slot]).wait()

        for b in range(_NBUF):
            start_gather(b, b)

        def body(g, _):
            for b in range(_NBUF):
                c = g * _NBUF + b
                wait_gather(c, b)
                start_write(c, b)
                wait_write(c, b)
                start_gather(c + _NBUF, b)
            return _

        lax.fori_loop(0, S // _NBUF - 1, body, None)

        for b in range(_NBUF):
            c = S - _NBUF + b
            wait_gather(c, b)
            start_write(c, b)
        for b in range(_NBUF):
            wait_write(S - _NBUF + b, b)

    return gather_kernel


def kernel(X, table):
    V, D = table.shape
    B = X.size
    assert B % (_NW * _CHUNK) == 0
    S = B // (_NW * _CHUNK)
    idx = X.reshape(_NW, S, _CHUNK).astype(jnp.int32)
    out = _build(V, D, S)(idx, table)
    return out.reshape(B, 2 * D)[:, :D].reshape(X.shape + (D,))
